# baseline (speedup 1.0000x reference)
import functools

import jax
import jax.numpy as jnp
from jax import lax
from jax.experimental import pallas as pl
from jax.experimental.pallas import tpu as pltpu
from jax.experimental.pallas import tpu_sc as plsc

_DIM = 64
_PAD = 128
_SCALE = float(_DIM) ** 0.5
_L = 16
_NC, _NS = 2, 16
_NW = _NC * _NS
_V = 1000000
_CPT = 246
_VLAST = (_V // _PAD - 1) * _PAD
_VTAIL = (_V // _PAD) * _PAD

_MESH = plsc.VectorSubcoreMesh(core_axis_name="c", subcore_axis_name="s")
_CP = pltpu.CompilerParams(use_tc_tiling_on_sc=True, needs_layout_passes=False)


@functools.partial(
    pl.kernel,
    out_type=jax.ShapeDtypeStruct((_V, _PAD), jnp.float32),
    mesh=_MESH,
    scratch_types=[
        pltpu.VMEM((_DIM, _PAD), jnp.float32),
        pltpu.VMEM((_DIM, _PAD), jnp.float32),
        pltpu.VMEM((_PAD, _PAD), jnp.float32),
        pltpu.VMEM((_PAD, _PAD), jnp.float32),
        pltpu.SemaphoreType.DMA,
        pltpu.SemaphoreType.DMA,
        pltpu.SemaphoreType.DMA,
        pltpu.SemaphoreType.DMA,
    ],
    compiler_params=_CP,
)
def _relayout(tabt_hbm, tail_hbm, scr_hbm, in0, in1, o0, o1, r0, r1, w0, w1):
    wid = lax.axis_index("s") * _NC + lax.axis_index("c")
    g0 = wid * _CPT
    d16_pad = [lax.iota(jnp.int32, _L) + tb * _L for tb in range(_PAD // _L)]

    def v0_of(c):
        return pl.multiple_of(jnp.minimum((g0 + c) * _PAD, _VLAST), _PAD)

    ins = (in0, in1)
    outs = (o0, o1)
    rsems = (r0, r1)
    wsems = (w0, w1)

    def fire_read(b, c):
        pltpu.async_copy(
            tabt_hbm.at[:, pl.ds(v0_of(c), _PAD)], ins[b], rsems[b]
        )

    fire_read(0, 0)
    fire_read(1, 1)

    def pair(p, carry):
        for b in range(2):
            c = 2 * p + b

            @pl.when(p > 0)
            def _():
                pltpu.make_async_copy(
                    outs[b], scr_hbm.at[pl.ds(v0_of(c), _PAD), :], wsems[b]
                ).wait()

            pltpu.make_async_copy(
                tabt_hbm.at[:, pl.ds(v0_of(c), _PAD)], ins[b], rsems[b]
            ).wait()

            @plsc.parallel_loop(0, _DIM, unroll=8)
            def per_d(d):
                ds_ = jnp.full((_L,), d, jnp.int32)
                for tb in range(_PAD // _L):
                    vec = ins[b][d, pl.ds(tb * _L, _L)]
                    plsc.store_scatter(outs[b], [d16_pad[tb], ds_], vec)
            pltpu.async_copy(
                outs[b], scr_hbm.at[pl.ds(v0_of(c), _PAD), :], wsems[b]
            )

            @pl.when(p < _CPT // 2 - 1)
            def _():
                fire_read(b, c + 2)

        return carry

    lax.fori_loop(0, _CPT // 2, pair, 0)
    for b in range(2):
        pltpu.make_async_copy(
            outs[b], scr_hbm.at[pl.ds(v0_of(_CPT - 2 + b), _PAD), :], wsems[b]
        ).wait()

    @pl.when(wid == 0)
    def _():
        pltpu.sync_copy(tail_hbm, scr_hbm.at[pl.ds(_VTAIL, _V - _VTAIL), :])


def _gather(ids_t, scr):
    n_slab, n_tok = ids_t.shape

    @functools.partial(
        pl.kernel,
        out_type=jax.ShapeDtypeStruct((n_slab, _DIM, n_tok), jnp.float32),
        mesh=_MESH,
        scratch_types=[
            pltpu.VMEM((n_slab, _PAD), jnp.int32),
            pltpu.VMEM((_PAD, _PAD), jnp.float32),
            pltpu.VMEM((_PAD, _PAD), jnp.float32),
            pltpu.VMEM((_DIM, _PAD), jnp.float32),
            pltpu.VMEM((_DIM, _PAD), jnp.float32),
            pltpu.SemaphoreType.DMA,
            pltpu.SemaphoreType.DMA,
            pltpu.SemaphoreType.DMA,
            pltpu.SemaphoreType.DMA,
        ],
        compiler_params=_CP,
    )
    def k(ids_hbm, scr_hbm, out_hbm, idx_all, rows0, rows1, blk0, blk1,
          g0, g1, w0, w1):
        wid = lax.axis_index("s") * _NC + lax.axis_index("c")
        col0 = wid * _PAD
        t16 = [lax.iota(jnp.int32, _L) + t * _L for t in range(_PAD // _L)]

        pltpu.sync_copy(ids_hbm.at[:, pl.ds(col0, _PAD)], idx_all)

        rows = (rows0, rows1)
        blks = (blk0, blk1)
        gsems = (g0, g1)
        wsems = (w0, w1)

        def fire_gather(b, c):
            pltpu.async_copy(scr_hbm.at[idx_all.at[c]], rows[b], gsems[b])

        fire_gather(0, 0)
        fire_gather(1, 1)

        def pair(p, carry):
            for b in range(2):
                c = 2 * p + b

                @pl.when(p > 0)
                def _():
                    pltpu.make_async_copy(
                        blks[b], out_hbm.at[c, :, pl.ds(col0, _PAD)], wsems[b]
                    ).wait()

                pltpu.make_async_copy(
                    scr_hbm.at[idx_all.at[c]], rows[b], gsems[b]
                ).wait()

                @plsc.parallel_loop(0, _PAD, unroll=8)
                def per_t(t):
                    ts_ = jnp.full((_L,), t, jnp.int32)
                    for db in range(_DIM // _L):
                        vec = rows[b][t, pl.ds(db * _L, _L)] * _SCALE
                        plsc.store_scatter(blks[b], [t16[db], ts_], vec)
                pltpu.async_copy(
                    blks[b], out_hbm.at[c, :, pl.ds(col0, _PAD)], wsems[b]
                )

                @pl.when(p < n_slab // 2 - 1)
                def _():
                    fire_gather(b, c + 2)

            return carry

        lax.fori_loop(0, n_slab // 2, pair, 0)
        for b in range(2):
            pltpu.make_async_copy(
                blks[b], out_hbm.at[n_slab - 2 + b, :, pl.ds(col0, _PAD)],
                wsems[b],
            ).wait()

    return k(ids_t, scr)


@jax.jit
def _embed(input_ids, table):
    tailp = jnp.pad(table[_VTAIL:], ((0, 0), (0, _PAD - _DIM)))
    scr = _relayout(table.T, tailp)
    out3 = _gather(input_ids.T, scr)
    return out3.transpose(2, 0, 1)


def kernel(input_ids, table):
    return _embed(input_ids, table)


# speedup vs baseline: 1.0029x; 1.0029x over previous
import functools

import jax
import jax.numpy as jnp
from jax import lax
from jax.experimental import pallas as pl
from jax.experimental.pallas import tpu as pltpu
from jax.experimental.pallas import tpu_sc as plsc

_DIM = 64
_PAD = 128
_SCALE = float(_DIM) ** 0.5
_L = 16
_NC, _NS = 2, 16
_NW = _NC * _NS
_V = 1000000
_CPT = 246
_VLAST = (_V // _PAD - 1) * _PAD
_VTAIL = (_V // _PAD) * _PAD

_MESH = plsc.VectorSubcoreMesh(core_axis_name="c", subcore_axis_name="s")
_CP = pltpu.CompilerParams(
    use_tc_tiling_on_sc=True,
    needs_layout_passes=False,
    disable_bounds_checks=True,
)


@functools.partial(
    pl.kernel,
    out_type=jax.ShapeDtypeStruct((_V, _PAD), jnp.float32),
    mesh=_MESH,
    scratch_types=[
        pltpu.VMEM((_DIM, _PAD), jnp.float32),
        pltpu.VMEM((_DIM, _PAD), jnp.float32),
        pltpu.VMEM((_PAD, _PAD), jnp.float32),
        pltpu.VMEM((_PAD, _PAD), jnp.float32),
        pltpu.SemaphoreType.DMA,
        pltpu.SemaphoreType.DMA,
        pltpu.SemaphoreType.DMA,
        pltpu.SemaphoreType.DMA,
    ],
    compiler_params=_CP,
)
def _relayout(tabt_hbm, tail_hbm, scr_hbm, in0, in1, o0, o1, r0, r1, w0, w1):
    wid = lax.axis_index("s") * _NC + lax.axis_index("c")
    g0 = wid * _CPT
    d16_pad = [lax.iota(jnp.int32, _L) + tb * _L for tb in range(_PAD // _L)]

    def v0_of(c):
        return pl.multiple_of(jnp.minimum((g0 + c) * _PAD, _VLAST), _PAD)

    ins = (in0, in1)
    outs = (o0, o1)
    rsems = (r0, r1)
    wsems = (w0, w1)

    def fire_read(b, c):
        pltpu.async_copy(
            tabt_hbm.at[:, pl.ds(v0_of(c), _PAD)], ins[b], rsems[b]
        )

    fire_read(0, 0)
    fire_read(1, 1)

    def pair(p, carry):
        for b in range(2):
            c = 2 * p + b

            @pl.when(p > 0)
            def _():
                pltpu.make_async_copy(
                    outs[b], scr_hbm.at[pl.ds(v0_of(c), _PAD), :], wsems[b]
                ).wait()

            pltpu.make_async_copy(
                tabt_hbm.at[:, pl.ds(v0_of(c), _PAD)], ins[b], rsems[b]
            ).wait()

            @plsc.parallel_loop(0, _DIM, unroll=8)
            def per_d(d):
                ds_ = jnp.full((_L,), d, jnp.int32)
                for tb in range(_PAD // _L):
                    vec = ins[b][d, pl.ds(tb * _L, _L)]
                    plsc.store_scatter(outs[b], [d16_pad[tb], ds_], vec)
            pltpu.async_copy(
                outs[b], scr_hbm.at[pl.ds(v0_of(c), _PAD), :], wsems[b]
            )

            @pl.when(p < _CPT // 2 - 1)
            def _():
                fire_read(b, c + 2)

        return carry

    lax.fori_loop(0, _CPT // 2, pair, 0)
    for b in range(2):
        pltpu.make_async_copy(
            outs[b], scr_hbm.at[pl.ds(v0_of(_CPT - 2 + b), _PAD), :], wsems[b]
        ).wait()

    @pl.when(wid == 0)
    def _():
        pltpu.sync_copy(tail_hbm, scr_hbm.at[pl.ds(_VTAIL, _V - _VTAIL), :])


def _gather(ids_t, scr):
    n_slab, n_tok = ids_t.shape

    @functools.partial(
        pl.kernel,
        out_type=jax.ShapeDtypeStruct((n_slab, _DIM, n_tok), jnp.float32),
        mesh=_MESH,
        scratch_types=[
            pltpu.VMEM((n_slab, _PAD), jnp.int32),
            pltpu.VMEM((_PAD, _PAD), jnp.float32),
            pltpu.VMEM((_PAD, _PAD), jnp.float32),
            pltpu.VMEM((_DIM, _PAD), jnp.float32),
            pltpu.VMEM((_DIM, _PAD), jnp.float32),
            pltpu.SemaphoreType.DMA,
            pltpu.SemaphoreType.DMA,
            pltpu.SemaphoreType.DMA,
            pltpu.SemaphoreType.DMA,
        ],
        compiler_params=_CP,
    )
    def k(ids_hbm, scr_hbm, out_hbm, idx_all, rows0, rows1, blk0, blk1,
          g0, g1, w0, w1):
        wid = lax.axis_index("s") * _NC + lax.axis_index("c")
        col0 = wid * _PAD
        t16 = [lax.iota(jnp.int32, _L) + t * _L for t in range(_PAD // _L)]

        pltpu.sync_copy(ids_hbm.at[:, pl.ds(col0, _PAD)], idx_all)

        rows = (rows0, rows1)
        blks = (blk0, blk1)
        gsems = (g0, g1)
        wsems = (w0, w1)

        def fire_gather(b, c):
            pltpu.async_copy(scr_hbm.at[idx_all.at[c]], rows[b], gsems[b])

        fire_gather(0, 0)
        fire_gather(1, 1)

        def pair(p, carry):
            for b in range(2):
                c = 2 * p + b

                @pl.when(p > 0)
                def _():
                    pltpu.make_async_copy(
                        blks[b], out_hbm.at[c, :, pl.ds(col0, _PAD)], wsems[b]
                    ).wait()

                pltpu.make_async_copy(
                    scr_hbm.at[idx_all.at[c]], rows[b], gsems[b]
                ).wait()

                @plsc.parallel_loop(0, _PAD, unroll=8)
                def per_t(t):
                    ts_ = jnp.full((_L,), t, jnp.int32)
                    for db in range(_DIM // _L):
                        vec = rows[b][t, pl.ds(db * _L, _L)] * _SCALE
                        plsc.store_scatter(blks[b], [t16[db], ts_], vec)
                pltpu.async_copy(
                    blks[b], out_hbm.at[c, :, pl.ds(col0, _PAD)], wsems[b]
                )

                @pl.when(p < n_slab // 2 - 1)
                def _():
                    fire_gather(b, c + 2)

            return carry

        lax.fori_loop(0, n_slab // 2, pair, 0)
        for b in range(2):
            pltpu.make_async_copy(
                blks[b], out_hbm.at[n_slab - 2 + b, :, pl.ds(col0, _PAD)],
                wsems[b],
            ).wait()

    return k(ids_t, scr)


@jax.jit
def _embed(input_ids, table):
    tailp = jnp.pad(table[_VTAIL:], ((0, 0), (0, _PAD - _DIM)))
    scr = _relayout(table.T, tailp)
    out3 = _gather(input_ids.T, scr)
    return out3.transpose(2, 0, 1)


def kernel(input_ids, table):
    return _embed(input_ids, table)


# speedup vs baseline: 1.5944x; 1.5898x over previous
import functools

import jax
import jax.numpy as jnp
from jax import lax
from jax.experimental import pallas as pl
from jax.experimental.pallas import tpu as pltpu
from jax.experimental.pallas import tpu_sc as plsc

_DIM = 64
_PAD = 128
_SCALE = float(_DIM) ** 0.5
_L = 16
_NC, _NS = 2, 16
_NW = _NC * _NS
_V = 1000000
_CPT = 246
_VLAST = (_V // _PAD - 1) * _PAD
_VTAIL = (_V // _PAD) * _PAD

_MESH = plsc.VectorSubcoreMesh(core_axis_name="c", subcore_axis_name="s")
_CP = pltpu.CompilerParams(
    use_tc_tiling_on_sc=True,
    needs_layout_passes=False,
    disable_bounds_checks=True,
)


@functools.partial(
    pl.kernel,
    out_type=jax.ShapeDtypeStruct((_V, _PAD), jnp.float32),
    mesh=_MESH,
    scratch_types=[
        pltpu.VMEM((_DIM, _PAD), jnp.float32),
        pltpu.VMEM((_DIM, _PAD), jnp.float32),
        pltpu.VMEM((_PAD, _PAD), jnp.float32),
        pltpu.VMEM((_PAD, _PAD), jnp.float32),
        pltpu.SemaphoreType.DMA,
        pltpu.SemaphoreType.DMA,
        pltpu.SemaphoreType.DMA,
        pltpu.SemaphoreType.DMA,
    ],
    compiler_params=_CP,
)
def _relayout(tabt_hbm, tail_hbm, scr_hbm, in0, in1, o0, o1, r0, r1, w0, w1):
    wid = lax.axis_index("s") * _NC + lax.axis_index("c")
    g0 = wid * _CPT
    iota = lax.iota(jnp.int32, _L)
    d16 = [iota + db * _L for db in range(_DIM // _L)]
    rot = [(iota + s) & (_L - 1) for s in range(_L)]

    def v0_of(c):
        return pl.multiple_of(jnp.minimum((g0 + c) * _PAD, _VLAST), _PAD)

    ins = (in0, in1)
    outs = (o0, o1)
    rsems = (r0, r1)
    wsems = (w0, w1)

    def fire_read(b, c):
        pltpu.async_copy(
            tabt_hbm.at[:, pl.ds(v0_of(c), _PAD)], ins[b], rsems[b]
        )

    fire_read(0, 0)
    fire_read(1, 1)

    def pair(p, carry):
        for b in range(2):
            c = 2 * p + b

            @pl.when(p > 0)
            def _():
                pltpu.make_async_copy(
                    outs[b], scr_hbm.at[pl.ds(v0_of(c), _PAD), :], wsems[b]
                ).wait()

            pltpu.make_async_copy(
                tabt_hbm.at[:, pl.ds(v0_of(c), _PAD)], ins[b], rsems[b]
            ).wait()

            for db in range(_DIM // _L):

                @plsc.parallel_loop(0, _PAD // _L, unroll=2)
                def per_vb(vb):
                    vbase = jnp.full((_L,), vb * _L, jnp.int32)
                    for s in range(_L):
                        v_idx = vbase + rot[s]
                        val = plsc.load_gather(ins[b], [d16[db], v_idx])
                        plsc.store_scatter(outs[b], [v_idx, d16[db]], val)
            pltpu.async_copy(
                outs[b], scr_hbm.at[pl.ds(v0_of(c), _PAD), :], wsems[b]
            )

            @pl.when(p < _CPT // 2 - 1)
            def _():
                fire_read(b, c + 2)

        return carry

    lax.fori_loop(0, _CPT // 2, pair, 0)
    for b in range(2):
        pltpu.make_async_copy(
            outs[b], scr_hbm.at[pl.ds(v0_of(_CPT - 2 + b), _PAD), :], wsems[b]
        ).wait()

    @pl.when(wid == 0)
    def _():
        pltpu.sync_copy(tail_hbm, scr_hbm.at[pl.ds(_VTAIL, _V - _VTAIL), :])


def _gather(ids_t, scr):
    n_slab, n_tok = ids_t.shape

    @functools.partial(
        pl.kernel,
        out_type=jax.ShapeDtypeStruct((n_slab, _DIM, n_tok), jnp.float32),
        mesh=_MESH,
        scratch_types=[
            pltpu.VMEM((n_slab, _PAD), jnp.int32),
            pltpu.VMEM((_PAD, _PAD), jnp.float32),
            pltpu.VMEM((_PAD, _PAD), jnp.float32),
            pltpu.VMEM((_DIM, _PAD), jnp.float32),
            pltpu.VMEM((_DIM, _PAD), jnp.float32),
            pltpu.SemaphoreType.DMA,
            pltpu.SemaphoreType.DMA,
            pltpu.SemaphoreType.DMA,
            pltpu.SemaphoreType.DMA,
        ],
        compiler_params=_CP,
    )
    def k(ids_hbm, scr_hbm, out_hbm, idx_all, rows0, rows1, blk0, blk1,
          g0, g1, w0, w1):
        wid = lax.axis_index("s") * _NC + lax.axis_index("c")
        col0 = wid * _PAD
        iota = lax.iota(jnp.int32, _L)
        d16 = [iota + db * _L for db in range(_DIM // _L)]
        rot = [(iota + s) & (_L - 1) for s in range(_L)]

        pltpu.sync_copy(ids_hbm.at[:, pl.ds(col0, _PAD)], idx_all)

        rows = (rows0, rows1)
        blks = (blk0, blk1)
        gsems = (g0, g1)
        wsems = (w0, w1)

        def fire_gather(b, c):
            pltpu.async_copy(scr_hbm.at[idx_all.at[c]], rows[b], gsems[b])

        fire_gather(0, 0)
        fire_gather(1, 1)

        def pair(p, carry):
            for b in range(2):
                c = 2 * p + b

                @pl.when(p > 0)
                def _():
                    pltpu.make_async_copy(
                        blks[b], out_hbm.at[c, :, pl.ds(col0, _PAD)], wsems[b]
                    ).wait()

                pltpu.make_async_copy(
                    scr_hbm.at[idx_all.at[c]], rows[b], gsems[b]
                ).wait()

                for db in range(_DIM // _L):

                    @plsc.parallel_loop(0, _PAD // _L, unroll=2)
                    def per_tb(tb):
                        tbase = jnp.full((_L,), tb * _L, jnp.int32)
                        for s in range(_L):
                            t_idx = tbase + rot[s]
                            val = plsc.load_gather(rows[b], [t_idx, d16[db]])
                            plsc.store_scatter(
                                blks[b], [d16[db], t_idx], val * _SCALE
                            )
                pltpu.async_copy(
                    blks[b], out_hbm.at[c, :, pl.ds(col0, _PAD)], wsems[b]
                )

                @pl.when(p < n_slab // 2 - 1)
                def _():
                    fire_gather(b, c + 2)

            return carry

        lax.fori_loop(0, n_slab // 2, pair, 0)
        for b in range(2):
            pltpu.make_async_copy(
                blks[b], out_hbm.at[n_slab - 2 + b, :, pl.ds(col0, _PAD)],
                wsems[b],
            ).wait()

    return k(ids_t, scr)


@jax.jit
def _embed(input_ids, table):
    tailp = jnp.pad(table[_VTAIL:], ((0, 0), (0, _PAD - _DIM)))
    scr = _relayout(table.T, tailp)
    out3 = _gather(input_ids.T, scr)
    return out3.transpose(2, 0, 1)


def kernel(input_ids, table):
    return _embed(input_ids, table)


# speedup vs baseline: 2.7547x; 1.7277x over previous
import functools

import jax
import jax.numpy as jnp
from jax import lax
from jax.experimental import pallas as pl
from jax.experimental.pallas import tpu as pltpu
from jax.experimental.pallas import tpu_sc as plsc

_DIM = 64
_PAD = 128
_SCALE = float(_DIM) ** 0.5
_L = 16
_NC, _NS = 2, 16
_NW = _NC * _NS
_V = 1000000
_CPT = 246
_VLAST = (_V // _PAD - 1) * _PAD
_VTAIL = (_V // _PAD) * _PAD

_MESH = plsc.VectorSubcoreMesh(core_axis_name="c", subcore_axis_name="s")
_CP = pltpu.CompilerParams(
    use_tc_tiling_on_sc=True,
    needs_layout_passes=False,
    disable_bounds_checks=True,
)


@functools.partial(
    pl.kernel,
    out_type=jax.ShapeDtypeStruct((_V, _PAD), jnp.float32),
    mesh=_MESH,
    scratch_types=[
        pltpu.VMEM((_DIM, _PAD), jnp.float32),
        pltpu.VMEM((_DIM, _PAD), jnp.float32),
        pltpu.VMEM((_PAD, _PAD), jnp.float32),
        pltpu.VMEM((_PAD, _PAD), jnp.float32),
        pltpu.SemaphoreType.DMA,
        pltpu.SemaphoreType.DMA,
        pltpu.SemaphoreType.DMA,
        pltpu.SemaphoreType.DMA,
    ],
    compiler_params=_CP,
)
def _relayout(tabt_hbm, tail_hbm, scr_hbm, in0, in1, o0, o1, r0, r1, w0, w1):
    wid = lax.axis_index("s") * _NC + lax.axis_index("c")
    g0 = wid * _CPT
    iota = lax.iota(jnp.int32, _L)
    d16 = [iota + db * _L for db in range(_DIM // _L)]
    rot = [(iota + s) & (_L - 1) for s in range(_L)]

    def v0_of(c):
        return pl.multiple_of(jnp.minimum((g0 + c) * _PAD, _VLAST), _PAD)

    ins = (in0, in1)
    outs = (o0, o1)
    rsems = (r0, r1)
    wsems = (w0, w1)

    def fire_read(b, c):
        pltpu.async_copy(
            tabt_hbm.at[:, pl.ds(v0_of(c), _PAD)], ins[b], rsems[b]
        )

    fire_read(0, 0)
    fire_read(1, 1)

    def pair(p, carry):
        for b in range(2):
            c = 2 * p + b

            @pl.when(p > 0)
            def _():
                pltpu.make_async_copy(
                    outs[b], scr_hbm.at[pl.ds(v0_of(c), _PAD), :], wsems[b]
                ).wait()

            pltpu.make_async_copy(
                tabt_hbm.at[:, pl.ds(v0_of(c), _PAD)], ins[b], rsems[b]
            ).wait()

            for db in range(_DIM // _L):

                @plsc.parallel_loop(0, _PAD // _L, unroll=4)
                def per_vb(vb):
                    vbase = jnp.full((_L,), vb * _L, jnp.int32)
                    for s in range(_L):
                        v_idx = vbase + rot[s]
                        val = plsc.load_gather(ins[b], [d16[db], v_idx])
                        plsc.store_scatter(outs[b], [v_idx, d16[db]], val)
            pltpu.async_copy(
                outs[b], scr_hbm.at[pl.ds(v0_of(c), _PAD), :], wsems[b]
            )

            @pl.when(p < _CPT // 2 - 1)
            def _():
                fire_read(b, c + 2)

        return carry

    lax.fori_loop(0, _CPT // 2, pair, 0)
    for b in range(2):
        pltpu.make_async_copy(
            outs[b], scr_hbm.at[pl.ds(v0_of(_CPT - 2 + b), _PAD), :], wsems[b]
        ).wait()

    @pl.when(wid == 0)
    def _():
        pltpu.sync_copy(tail_hbm, scr_hbm.at[pl.ds(_VTAIL, _V - _VTAIL), :])


def _gather(ids_t, scr):
    n_slab, n_tok = ids_t.shape

    @functools.partial(
        pl.kernel,
        out_type=jax.ShapeDtypeStruct((n_slab, _DIM, n_tok), jnp.float32),
        mesh=_MESH,
        scratch_types=[
            pltpu.VMEM((n_slab, _PAD), jnp.int32),
            pltpu.VMEM((_PAD, _PAD), jnp.float32),
            pltpu.VMEM((_PAD, _PAD), jnp.float32),
            pltpu.VMEM((_DIM, _PAD), jnp.float32),
            pltpu.VMEM((_DIM, _PAD), jnp.float32),
            pltpu.SemaphoreType.DMA,
            pltpu.SemaphoreType.DMA,
            pltpu.SemaphoreType.DMA,
            pltpu.SemaphoreType.DMA,
        ],
        compiler_params=_CP,
    )
    def k(ids_hbm, scr_hbm, out_hbm, idx_all, rows0, rows1, blk0, blk1,
          g0, g1, w0, w1):
        wid = lax.axis_index("s") * _NC + lax.axis_index("c")
        col0 = wid * _PAD
        iota = lax.iota(jnp.int32, _L)
        d16 = [iota + db * _L for db in range(_DIM // _L)]
        rot = [(iota + s) & (_L - 1) for s in range(_L)]

        pltpu.sync_copy(ids_hbm.at[:, pl.ds(col0, _PAD)], idx_all)

        rows = (rows0, rows1)
        blks = (blk0, blk1)
        gsems = (g0, g1)
        wsems = (w0, w1)

        def fire_gather(b, c):
            pltpu.async_copy(scr_hbm.at[idx_all.at[c]], rows[b], gsems[b])

        fire_gather(0, 0)
        fire_gather(1, 1)

        def pair(p, carry):
            for b in range(2):
                c = 2 * p + b

                @pl.when(p > 0)
                def _():
                    pltpu.make_async_copy(
                        blks[b], out_hbm.at[c, :, pl.ds(col0, _PAD)], wsems[b]
                    ).wait()

                pltpu.make_async_copy(
                    scr_hbm.at[idx_all.at[c]], rows[b], gsems[b]
                ).wait()

                for db in range(_DIM // _L):

                    @plsc.parallel_loop(0, _PAD // _L, unroll=4)
                    def per_tb(tb):
                        tbase = jnp.full((_L,), tb * _L, jnp.int32)
                        for s in range(_L):
                            t_idx = tbase + rot[s]
                            val = plsc.load_gather(rows[b], [t_idx, d16[db]])
                            plsc.store_scatter(
                                blks[b], [d16[db], t_idx], val * _SCALE
                            )
                pltpu.async_copy(
                    blks[b], out_hbm.at[c, :, pl.ds(col0, _PAD)], wsems[b]
                )

                @pl.when(p < n_slab // 2 - 1)
                def _():
                    fire_gather(b, c + 2)

            return carry

        lax.fori_loop(0, n_slab // 2, pair, 0)
        for b in range(2):
            pltpu.make_async_copy(
                blks[b], out_hbm.at[n_slab - 2 + b, :, pl.ds(col0, _PAD)],
                wsems[b],
            ).wait()

    return k(ids_t, scr)


@jax.jit
def _embed(input_ids, table):
    tailp = jnp.pad(table[_VTAIL:], ((0, 0), (0, _PAD - _DIM)))
    scr = _relayout(table.T, tailp)
    out3 = _gather(input_ids.T, scr)
    return out3.transpose(2, 0, 1)


def kernel(input_ids, table):
    return _embed(input_ids, table)
